# baseline (speedup 1.0000x reference)
import jax
import jax.numpy as jnp
from jax.experimental import pallas as pl
from jax.experimental.pallas import tpu as pltpu


def _per_channel_kernel(x_ref, w_ref, b_ref, o_ref):
    y = jax.lax.dot_general(
        x_ref[...], w_ref[...].astype(jnp.bfloat16),
        dimension_numbers=(((2,), (1,)), ((1,), (0,))),
        preferred_element_type=jnp.float32)
    o_ref[...] = (y + b_ref[...][:, None, :]).astype(o_ref.dtype)


def kernel(encoded, weight, bias, *, tile_b=2048):
    B, C, H = encoded.shape
    Cw, Hw, F = weight.shape
    assert (C, H) == (Cw, Hw) and bias.shape == (C, F)

    tb = min(tile_b, B)
    pad = (-B) % tb
    if pad:
        encoded = jnp.pad(encoded, ((0, pad), (0, 0), (0, 0)))
    Bp = encoded.shape[0]
    encoded = encoded.astype(jnp.bfloat16)

    out_cbf = pl.pallas_call(
        _per_channel_kernel,
        out_shape=jax.ShapeDtypeStruct((C, Bp, F), jnp.bfloat16),
        grid=(Bp // tb,),
        in_specs=[
            pl.BlockSpec((tb, C, H), lambda i: (i, 0, 0)),
            pl.BlockSpec((C, H, F), lambda i: (0, 0, 0)),
            pl.BlockSpec((C, F), lambda i: (0, 0)),
        ],
        out_specs=pl.BlockSpec((C, tb, F), lambda i: (0, i, 0)),
        compiler_params=pltpu.CompilerParams(
            dimension_semantics=("parallel",)),
    )(encoded, weight, bias)

    out = jnp.transpose(out_cbf, (1, 2, 0)).astype(encoded.dtype)
    return out[:B]


# speedup vs baseline: 1.1268x; 1.1268x over previous
import jax
import jax.numpy as jnp
from jax.experimental import pallas as pl
from jax.experimental.pallas import tpu as pltpu


def _per_channel_kernel(x_ref, w_ref, b_ref, o_ref):
    y = jax.lax.dot_general(
        x_ref[...], w_ref[...],
        dimension_numbers=(((2,), (1,)), ((1,), (0,))),
        preferred_element_type=jnp.float32)
    o_ref[...] = (y + b_ref[...][:, None, :]).astype(o_ref.dtype)


def kernel(encoded, weight, bias, *, tile_b=2048):
    B, C, H = encoded.shape
    Cw, Hw, F = weight.shape
    assert (C, H) == (Cw, Hw) and bias.shape == (C, F)

    tb = min(tile_b, B)
    pad = (-B) % tb
    if pad:
        encoded = jnp.pad(encoded, ((0, pad), (0, 0), (0, 0)))
    Bp = encoded.shape[0]

    out_cbf = pl.pallas_call(
        _per_channel_kernel,
        out_shape=jax.ShapeDtypeStruct((C, Bp, F), jnp.bfloat16),
        grid=(Bp // tb,),
        in_specs=[
            pl.BlockSpec((tb, C, H), lambda i: (i, 0, 0)),
            pl.BlockSpec((C, H, F), lambda i: (0, 0, 0)),
            pl.BlockSpec((C, F), lambda i: (0, 0)),
        ],
        out_specs=pl.BlockSpec((C, tb, F), lambda i: (0, i, 0)),
        compiler_params=pltpu.CompilerParams(
            dimension_semantics=("parallel",)),
    )(encoded, weight, bias)

    out = jnp.transpose(out_cbf, (1, 2, 0)).astype(encoded.dtype)
    return out[:B]
